# Initial kernel scaffold; baseline (speedup 1.0000x reference)
import jax
import jax.numpy as jnp
from jax.experimental import pallas as pl


def kernel(x, adj, W1, b1, W2, b2):
    raise NotImplementedError("write your pallas kernel here")


# speedup vs baseline: 6.1536x; 6.1536x over previous
import functools
import jax
import jax.numpy as jnp
from jax import lax
from jax.experimental import pallas as pl
from jax.experimental.pallas import tpu as pltpu
from jax.experimental.pallas import tpu_sc as plsc

N_NODES = 10000
N_PAD = 10240
N_EDGES = 320000
D = 64
NC = 2
NS = 16
NW = NC * NS
EDGES_PER_WORKER = N_EDGES // NW
CHUNK = 80
NCHUNKS = EDGES_PER_WORKER // CHUNK
ROWS_PER_SUB = N_PAD // NS
ZROWS = 128


def _sc_gather_scatter(tables, src3, dst3, nt):
  mesh = plsc.VectorSubcoreMesh(core_axis_name="c", subcore_axis_name="s")

  def body(tab_hbm, src_hbm, dst_hbm, out_hbm, src_v, dst_v, rows_v, zbuf,
           acc_sh, sem):
    c = lax.axis_index("c")
    s = lax.axis_index("s")
    wid = c * NS + s
    row0 = pl.multiple_of(s * ROWS_PER_SUB, 8)

    pltpu.sync_copy(src_hbm.at[wid], src_v)
    pltpu.sync_copy(dst_hbm.at[wid], dst_v)

    zero = jnp.zeros((16,), jnp.float32)
    nsub = D // 16

    def zstore(i, carry):
      r = i // nsub
      col = (i % nsub) * 16
      zbuf[r, pl.ds(col, 16)] = zero
      return carry

    lax.fori_loop(0, ZROWS * nsub, zstore, 0)

    for t in range(nt):
      for z in range(ROWS_PER_SUB // ZROWS):
        pltpu.sync_copy(zbuf, acc_sh.at[pl.ds(row0 + z * ZROWS, ZROWS)])
      plsc.subcore_barrier()

      def chunk_body(j, carry):
        pltpu.async_copy(tab_hbm.at[t].at[src_v.at[j]], rows_v, sem).wait()
        pltpu.sync_copy(rows_v, acc_sh.at[dst_v.at[j]], add=True)
        return carry

      lax.fori_loop(0, NCHUNKS, chunk_body, 0)
      plsc.subcore_barrier()

      pltpu.sync_copy(acc_sh.at[pl.ds(row0, ROWS_PER_SUB)],
                      out_hbm.at[t, c, pl.ds(row0, ROWS_PER_SUB)])
      plsc.subcore_barrier()

  k = pl.kernel(
      body,
      out_type=jax.ShapeDtypeStruct((nt, NC, N_PAD, D), jnp.float32),
      mesh=mesh,
      compiler_params=pltpu.CompilerParams(use_tc_tiling_on_sc=False),
      scratch_types=[
          pltpu.VMEM((NCHUNKS, CHUNK), jnp.int32),
          pltpu.VMEM((NCHUNKS, CHUNK), jnp.int32),
          pltpu.VMEM((CHUNK, D), jnp.float32),
          pltpu.VMEM((ZROWS, D), jnp.float32),
          pltpu.VMEM_SHARED((N_PAD, D), jnp.float32),
          pltpu.SemaphoreType.DMA,
      ],
  )
  return k(tables, src3, dst3)


def _tc_matmul_split(x, w):
  bm = 1000
  n, kin = x.shape
  ws = jnp.stack([w[:, :D], w[:, D:]])

  def body(x_ref, w_ref, o_ref):
    o_ref[0] = jnp.dot(x_ref[...], w_ref[0],
                       preferred_element_type=jnp.float32)

  return pl.pallas_call(
      body,
      grid=(2, n // bm),
      in_specs=[
          pl.BlockSpec((bm, kin), lambda j, i: (i, 0)),
          pl.BlockSpec((1, kin, D), lambda j, i: (j, 0, 0)),
      ],
      out_specs=pl.BlockSpec((1, bm, D), lambda j, i: (j, i, 0)),
      out_shape=jax.ShapeDtypeStruct((2, n, D), jnp.float32),
  )(x, ws)


def _tc_fuse_relu_matmul(parts, b, w):
  bm = 1000
  kout = w.shape[1]

  def body(p_ref, b_ref, w_ref, o_ref):
    p = p_ref[...]
    h = jnp.concatenate([p[0, 0] + p[0, 1], p[1, 0] + p[1, 1]], axis=-1)
    h = jax.nn.relu(h + b_ref[...])
    o_ref[...] = jnp.dot(h, w_ref[...], preferred_element_type=jnp.float32)

  return pl.pallas_call(
      body,
      grid=(N_NODES // bm,),
      in_specs=[
          pl.BlockSpec((2, NC, bm, D), lambda i: (0, 0, i, 0)),
          pl.BlockSpec((1, 2 * D), lambda i: (0, 0)),
          pl.BlockSpec((2 * D, kout), lambda i: (0, 0)),
      ],
      out_specs=pl.BlockSpec((bm, kout), lambda i: (i, 0)),
      out_shape=jax.ShapeDtypeStruct((N_NODES, kout), jnp.float32),
  )(parts, b.reshape(1, 2 * D), w)


def _tc_sum_bias(parts, b):
  bm = 1000

  def body(p_ref, b_ref, o_ref):
    p = p_ref[...]
    o_ref[...] = p[0, 0] + p[0, 1] + b_ref[...]

  return pl.pallas_call(
      body,
      grid=(N_NODES // bm,),
      in_specs=[
          pl.BlockSpec((1, NC, bm, D), lambda i: (0, 0, i, 0)),
          pl.BlockSpec((1, D), lambda i: (0, 0)),
      ],
      out_specs=pl.BlockSpec((bm, D), lambda i: (i, 0)),
      out_shape=jax.ShapeDtypeStruct((N_NODES, D), jnp.float32),
  )(parts, b.reshape(1, D))


def kernel(x, adj, W1, b1, W2, b2):
  src3 = adj[0].reshape(NW, NCHUNKS, CHUNK)
  dst3 = adj[1].reshape(NW, NCHUNKS, CHUNK)

  h = _tc_matmul_split(x, W1)
  parts1 = _sc_gather_scatter(h, src3, dst3, 2)
  g = _tc_fuse_relu_matmul(parts1, b1, W2)
  parts2 = _sc_gather_scatter(g[None], src3, dst3, 1)
  return _tc_sum_bias(parts2, b2)


# speedup vs baseline: 12.3757x; 2.0111x over previous
import functools
import jax
import jax.numpy as jnp
from jax import lax
from jax.experimental import pallas as pl
from jax.experimental.pallas import tpu as pltpu
from jax.experimental.pallas import tpu_sc as plsc

N_NODES = 10000
N_PAD = 10240
N_EDGES = 320000
D = 64
NC = 2
NS = 16
NW = NC * NS
EDGES_PER_WORKER = N_EDGES // NW
CHUNK = 125
NCHUNKS = EDGES_PER_WORKER // CHUNK
NBUF = 4
ROWS_PER_SUB = N_PAD // NS
ZROWS = 128


def _sc_gather_scatter(tables, src3, dst3, nt):
  mesh = plsc.VectorSubcoreMesh(core_axis_name="c", subcore_axis_name="s")

  def body(tab_hbm, src_hbm, dst_hbm, out_hbm, src_v, dst_v, r0, r1, r2, r3,
           zbuf, acc_sh, s0, s1, s2, s3):
    rows_vs = [r0, r1, r2, r3]
    sems = [s0, s1, s2, s3]
    c = lax.axis_index("c")
    s = lax.axis_index("s")
    wid = c * NS + s
    row0 = pl.multiple_of(s * ROWS_PER_SUB, 8)

    pltpu.sync_copy(src_hbm.at[wid], src_v)
    pltpu.sync_copy(dst_hbm.at[wid], dst_v)

    zero = jnp.zeros((16,), jnp.float32)
    nsub = D // 16

    def zstore(i, carry):
      r = i // nsub
      col = (i % nsub) * 16
      zbuf[r, pl.ds(col, 16)] = zero
      return carry

    lax.fori_loop(0, ZROWS * nsub, zstore, 0)

    for t in range(nt):
      for z in range(ROWS_PER_SUB // ZROWS):
        pltpu.sync_copy(zbuf, acc_sh.at[pl.ds(row0 + z * ZROWS, ZROWS)])
      plsc.subcore_barrier()

      for b in range(NBUF):
        pltpu.async_copy(tab_hbm.at[t].at[src_v.at[b]], rows_vs[b], sems[b])

      def group_body(g, carry):
        for b in range(NBUF):
          j = g * NBUF + b
          pltpu.make_async_copy(tab_hbm.at[t].at[src_v.at[j]], rows_vs[b],
                                sems[b]).wait()
          pltpu.sync_copy(rows_vs[b], acc_sh.at[dst_v.at[j]], add=True)
          jn = j + NBUF

          @pl.when(jn < NCHUNKS)
          def _():
            pltpu.async_copy(tab_hbm.at[t].at[src_v.at[jn]], rows_vs[b],
                             sems[b])
        return carry

      lax.fori_loop(0, NCHUNKS // NBUF, group_body, 0)
      plsc.subcore_barrier()

      pltpu.sync_copy(acc_sh.at[pl.ds(row0, ROWS_PER_SUB)],
                      out_hbm.at[t, c, pl.ds(row0, ROWS_PER_SUB)])
      plsc.subcore_barrier()

  k = pl.kernel(
      body,
      out_type=jax.ShapeDtypeStruct((nt, NC, N_PAD, D), jnp.float32),
      mesh=mesh,
      compiler_params=pltpu.CompilerParams(use_tc_tiling_on_sc=False),
      scratch_types=[
          pltpu.VMEM((NCHUNKS, CHUNK), jnp.int32),
          pltpu.VMEM((NCHUNKS, CHUNK), jnp.int32),
          pltpu.VMEM((CHUNK, D), jnp.float32),
          pltpu.VMEM((CHUNK, D), jnp.float32),
          pltpu.VMEM((CHUNK, D), jnp.float32),
          pltpu.VMEM((CHUNK, D), jnp.float32),
          pltpu.VMEM((ZROWS, D), jnp.float32),
          pltpu.VMEM_SHARED((N_PAD, D), jnp.float32),
          pltpu.SemaphoreType.DMA,
          pltpu.SemaphoreType.DMA,
          pltpu.SemaphoreType.DMA,
          pltpu.SemaphoreType.DMA,
      ],
  )
  return k(tables, src3, dst3)


def _tc_matmul_split(x, w):
  bm = 1000
  n, kin = x.shape
  ws = jnp.stack([w[:, :D], w[:, D:]])

  def body(x_ref, w_ref, o_ref):
    o_ref[0] = jnp.dot(x_ref[...], w_ref[0],
                       preferred_element_type=jnp.float32)

  return pl.pallas_call(
      body,
      grid=(2, n // bm),
      in_specs=[
          pl.BlockSpec((bm, kin), lambda j, i: (i, 0)),
          pl.BlockSpec((1, kin, D), lambda j, i: (j, 0, 0)),
      ],
      out_specs=pl.BlockSpec((1, bm, D), lambda j, i: (j, i, 0)),
      out_shape=jax.ShapeDtypeStruct((2, n, D), jnp.float32),
  )(x, ws)


def _tc_fuse_relu_matmul(parts, b, w):
  bm = 1000
  kout = w.shape[1]

  def body(p_ref, b_ref, w_ref, o_ref):
    p = p_ref[...]
    h = jnp.concatenate([p[0, 0] + p[0, 1], p[1, 0] + p[1, 1]], axis=-1)
    h = jax.nn.relu(h + b_ref[...])
    o_ref[...] = jnp.dot(h, w_ref[...], preferred_element_type=jnp.float32)

  return pl.pallas_call(
      body,
      grid=(N_NODES // bm,),
      in_specs=[
          pl.BlockSpec((2, NC, bm, D), lambda i: (0, 0, i, 0)),
          pl.BlockSpec((1, 2 * D), lambda i: (0, 0)),
          pl.BlockSpec((2 * D, kout), lambda i: (0, 0)),
      ],
      out_specs=pl.BlockSpec((bm, kout), lambda i: (i, 0)),
      out_shape=jax.ShapeDtypeStruct((N_NODES, kout), jnp.float32),
  )(parts, b.reshape(1, 2 * D), w)


def _tc_sum_bias(parts, b):
  bm = 1000

  def body(p_ref, b_ref, o_ref):
    p = p_ref[...]
    o_ref[...] = p[0, 0] + p[0, 1] + b_ref[...]

  return pl.pallas_call(
      body,
      grid=(N_NODES // bm,),
      in_specs=[
          pl.BlockSpec((1, NC, bm, D), lambda i: (0, 0, i, 0)),
          pl.BlockSpec((1, D), lambda i: (0, 0)),
      ],
      out_specs=pl.BlockSpec((bm, D), lambda i: (i, 0)),
      out_shape=jax.ShapeDtypeStruct((N_NODES, D), jnp.float32),
  )(parts, b.reshape(1, D))


def kernel(x, adj, W1, b1, W2, b2):
  src3 = adj[0].reshape(NW, NCHUNKS, CHUNK)
  dst3 = adj[1].reshape(NW, NCHUNKS, CHUNK)

  h = _tc_matmul_split(x, W1)
  parts1 = _sc_gather_scatter(h, src3, dst3, 2)
  g = _tc_fuse_relu_matmul(parts1, b1, W2)
  parts2 = _sc_gather_scatter(g[None], src3, dst3, 1)
  return _tc_sum_bias(parts2, b2)
